# baseline (speedup 1.0000x reference)
import functools

import jax
import jax.numpy as jnp
from jax import lax
from jax.experimental import pallas as pl
from jax.experimental.pallas import tpu as pltpu
from jax.experimental.pallas import tpu_sc as plsc

E = 64
NW = 32
CH = 128
K = 2
CHUNK = CH * K
VB = 32768


def _pack_table_tc(table_t, eye):
    V = table_t.shape[1]
    grid = (V + VB - 1) // VB
    H = VB // 2

    def body(x_ref, e_ref, o_ref):
        x = x_ref[...]
        eye_m = e_ref[...]
        lo = lax.dot_general(
            x[:, :H], eye_m, (((0,), (0,)), ((), ())),
            preferred_element_type=jnp.float32,
        )
        hi = lax.dot_general(
            x[:, H:], eye_m, (((0,), (0,)), ((), ())),
            preferred_element_type=jnp.float32,
        )
        o_ref[...] = jnp.concatenate([lo, hi], axis=1)

    return pl.pallas_call(
        body,
        grid=(grid,),
        in_specs=[
            pl.BlockSpec((E, VB), lambda j: (0, j)),
            pl.BlockSpec((E, E), lambda j: (0, 0)),
        ],
        out_specs=pl.BlockSpec((H, 2 * E), lambda j: (j, 0)),
        out_shape=jax.ShapeDtypeStruct((grid * H, 2 * E), jnp.float32),
        compiler_params=pltpu.CompilerParams(
            fuse_transposed_lhs_in_matmul=True),
    )(table_t, eye)


def _gather_sc(table2, idx, hb):
    N = idx.shape[0]
    b_per_w = N // NW
    n_chunks = b_per_w // CHUNK
    mesh = plsc.VectorSubcoreMesh(core_axis_name="c", subcore_axis_name="s")

    @functools.partial(
        pl.kernel,
        mesh=mesh,
        out_type=jax.ShapeDtypeStruct((N, E), jnp.float32),
        compiler_params=pltpu.CompilerParams(needs_layout_passes=False),
        scratch_types=[
            pltpu.VMEM((CHUNK,), jnp.int32),
            pltpu.VMEM((CHUNK,), jnp.int32),
            pltpu.VMEM((CHUNK, 2 * E), jnp.float32),
            pltpu.VMEM((CHUNK, E), jnp.float32),
            pltpu.SemaphoreType.DMA,
        ],
    )
    def k(table_hbm, idx_hbm, hb_hbm, out_hbm, idx_v, hb_v, rows_v, sel_v,
          sem):
        c = lax.axis_index("c")
        s = lax.axis_index("s")
        wid = s * 2 + c
        base = wid * b_per_w

        def body(j, carry):
            off = base + j * CHUNK
            pltpu.sync_copy(idx_hbm.at[pl.ds(off, CHUNK)], idx_v)
            pltpu.sync_copy(hb_hbm.at[pl.ds(off, CHUNK)], hb_v)
            copies = []
            for t in range(K):
                copies.append(
                    pltpu.async_copy(
                        table_hbm.at[idx_v.at[pl.ds(t * CH, CH)]],
                        rows_v.at[pl.ds(t * CH, CH)],
                        sem,
                    )
                )
            for cp in copies:
                cp.wait()

            def sel_group(g, carry2):
                rows16 = g * 16 + lax.iota(jnp.int32, 16)
                colb = hb_v[pl.ds(g * 16, 16)] * E
                for kk in range(E):
                    v = plsc.load_gather(rows_v, [rows16, colb + kk])
                    plsc.store_scatter(
                        sel_v, [rows16, jnp.full((16,), kk, jnp.int32)], v)
                return carry2

            lax.fori_loop(0, CHUNK // 16, sel_group, 0)
            pltpu.sync_copy(sel_v, out_hbm.at[pl.ds(off, CHUNK)])
            return carry

        lax.fori_loop(0, n_chunks, body, 0)

    return k(table2, idx, hb)


def _project_tc(x3, w, S, B):
    def body(x_ref, w_ref, o_ref):
        o_ref[0] = lax.dot_general(
            w_ref[...], x_ref[0], (((1,), (1,)), ((), ())),
            preferred_element_type=jnp.float32,
        )

    return pl.pallas_call(
        body,
        grid=(S,),
        in_specs=[
            pl.BlockSpec((1, B, E), lambda s: (s, 0, 0)),
            pl.BlockSpec((E, E), lambda s: (0, 0)),
        ],
        out_specs=pl.BlockSpec((1, E, B), lambda s: (s, 0, 0)),
        out_shape=jax.ShapeDtypeStruct((S, E, B), jnp.float32),
    )(x3, w)


def kernel(sent1, sent2, embedding_table, W):
    B, S = sent1.shape
    H = VB // 2
    table2 = _pack_table_tc(embedding_table.T, jnp.eye(E, dtype=jnp.float32))
    i1 = sent1.T.astype(jnp.int32)
    i2 = sent2.T.astype(jnp.int32)
    sh = VB.bit_length() - 1

    def pack_idx(i):
        blk = i >> sh
        o = i & (VB - 1)
        return (blk << (sh - 1)) | (o & (H - 1)), (o >> (sh - 1)) & 1

    r1, h1 = pack_idx(i1)
    r2, h2 = pack_idx(i2)
    g1 = _gather_sc(table2, r1.reshape(-1), h1.reshape(-1))
    g2 = _gather_sc(table2, r2.reshape(-1), h2.reshape(-1))
    y1 = _project_tc(g1.reshape(S, B, E), W, S, B)
    y2 = _project_tc(g2.reshape(S, B, E), W, S, B)
    s1 = y1.transpose(2, 0, 1)
    s2 = y2.transpose(2, 0, 1)
    return (s1, s2)


# speedup vs baseline: 2.8919x; 2.8919x over previous
import functools

import jax
import jax.numpy as jnp
from jax import lax
from jax.experimental import pallas as pl
from jax.experimental.pallas import tpu as pltpu
from jax.experimental.pallas import tpu_sc as plsc

E = 64
NW = 32
CH = 128
K = 5
CHUNK = CH * K
VB = 32768


def _pack_table_tc(table_t, eye):
    V = table_t.shape[1]
    grid = (V + VB - 1) // VB
    H = VB // 2

    def body(x_ref, e_ref, o_ref):
        x = x_ref[...]
        eye_m = e_ref[...]
        lo = lax.dot_general(
            x[:, :H], eye_m, (((0,), (0,)), ((), ())),
            preferred_element_type=jnp.float32,
        )
        hi = lax.dot_general(
            x[:, H:], eye_m, (((0,), (0,)), ((), ())),
            preferred_element_type=jnp.float32,
        )
        o_ref[...] = jnp.concatenate([lo, hi], axis=1)

    return pl.pallas_call(
        body,
        grid=(grid,),
        in_specs=[
            pl.BlockSpec((E, VB), lambda j: (0, j)),
            pl.BlockSpec((E, E), lambda j: (0, 0)),
        ],
        out_specs=pl.BlockSpec((H, 2 * E), lambda j: (j, 0)),
        out_shape=jax.ShapeDtypeStruct((grid * H, 2 * E), jnp.float32),
        compiler_params=pltpu.CompilerParams(
            fuse_transposed_lhs_in_matmul=True),
    )(table_t, eye)


def _gather_sc(table2, idx):
    N = idx.shape[0]
    b_per_w = N // NW
    n_chunks = b_per_w // CHUNK
    mesh = plsc.VectorSubcoreMesh(core_axis_name="c", subcore_axis_name="s")

    @functools.partial(
        pl.kernel,
        mesh=mesh,
        out_type=jax.ShapeDtypeStruct((N, 2 * E), jnp.float32),
        scratch_types=[
            pltpu.VMEM((CHUNK,), jnp.int32),
            pltpu.VMEM((CHUNK, 2 * E), jnp.float32),
            pltpu.SemaphoreType.DMA,
        ],
    )
    def k(table_hbm, idx_hbm, out_hbm, idx_v, rows_v, sem):
        c = lax.axis_index("c")
        s = lax.axis_index("s")
        wid = s * 2 + c
        base = wid * b_per_w

        def body(j, carry):
            off = base + j * CHUNK
            pltpu.sync_copy(idx_hbm.at[pl.ds(off, CHUNK)], idx_v)
            copies = []
            for t in range(K):
                copies.append(
                    pltpu.async_copy(
                        table_hbm.at[idx_v.at[pl.ds(t * CH, CH)]],
                        rows_v.at[pl.ds(t * CH, CH)],
                        sem,
                    )
                )
            for cp in copies:
                cp.wait()
            pltpu.sync_copy(rows_v, out_hbm.at[pl.ds(off, CHUNK)])
            return carry

        lax.fori_loop(0, n_chunks, body, 0)

    return k(table2, idx)


def _project_tc(x3, w2, par, S, B):
    def body(x_ref, w_ref, p_ref, o_ref):
        z = lax.dot_general(
            w_ref[...], x_ref[0], (((1,), (1,)), ((), ())),
            preferred_element_type=jnp.float32,
        )
        zlo = z[:E, :]
        zhi = z[E:, :]
        o_ref[0] = zlo + p_ref[0] * (zhi - zlo)

    return pl.pallas_call(
        body,
        grid=(S,),
        in_specs=[
            pl.BlockSpec((1, B, 2 * E), lambda s: (s, 0, 0)),
            pl.BlockSpec((2 * E, 2 * E), lambda s: (0, 0)),
            pl.BlockSpec((1, 1, B), lambda s: (s, 0, 0)),
        ],
        out_specs=pl.BlockSpec((1, E, B), lambda s: (s, 0, 0)),
        out_shape=jax.ShapeDtypeStruct((S, E, B), jnp.float32),
    )(x3, w2, par)


def kernel(sent1, sent2, embedding_table, W):
    B, S = sent1.shape
    H = VB // 2
    table2 = _pack_table_tc(embedding_table.T, jnp.eye(E, dtype=jnp.float32))
    i1 = sent1.T.astype(jnp.int32)
    i2 = sent2.T.astype(jnp.int32)
    sh = VB.bit_length() - 1

    def pack_idx(i):
        blk = i >> sh
        o = i & (VB - 1)
        return (blk << (sh - 1)) | (o & (H - 1)), (o >> (sh - 1)) & 1

    r1, h1 = pack_idx(i1)
    r2, h2 = pack_idx(i2)
    zero = jnp.zeros((E, E), jnp.float32)
    w2 = jnp.block([[W, zero], [zero, W]])
    p1 = h1.astype(jnp.float32).reshape(S, 1, B)
    p2 = h2.astype(jnp.float32).reshape(S, 1, B)
    g1 = _gather_sc(table2, r1.reshape(-1))
    g2 = _gather_sc(table2, r2.reshape(-1))
    y1 = _project_tc(g1.reshape(S, B, 2 * E), w2, p1, S, B)
    y2 = _project_tc(g2.reshape(S, B, 2 * E), w2, p2, S, B)
    s1 = y1.transpose(2, 0, 1)
    s2 = y2.transpose(2, 0, 1)
    return (s1, s2)


# speedup vs baseline: 2.9017x; 1.0034x over previous
import functools

import jax
import jax.numpy as jnp
from jax import lax
from jax.experimental import pallas as pl
from jax.experimental.pallas import tpu as pltpu
from jax.experimental.pallas import tpu_sc as plsc

E = 64
NW = 32
CH = 128
K = 5
CHUNK = CH * K
VB = 32768


def _pack_table_tc(table_t, eye):
    V = table_t.shape[1]
    grid = (V + VB - 1) // VB
    H = VB // 2

    def body(x_ref, e_ref, o_ref):
        x = x_ref[...]
        eye_m = e_ref[...]
        lo = lax.dot_general(
            x[:, :H], eye_m, (((0,), (0,)), ((), ())),
            preferred_element_type=jnp.float32,
        )
        hi = lax.dot_general(
            x[:, H:], eye_m, (((0,), (0,)), ((), ())),
            preferred_element_type=jnp.float32,
        )
        o_ref[...] = jnp.concatenate([lo, hi], axis=1)

    return pl.pallas_call(
        body,
        grid=(grid,),
        in_specs=[
            pl.BlockSpec((E, VB), lambda j: (0, j)),
            pl.BlockSpec((E, E), lambda j: (0, 0)),
        ],
        out_specs=pl.BlockSpec((H, 2 * E), lambda j: (j, 0)),
        out_shape=jax.ShapeDtypeStruct((grid * H, 2 * E), jnp.float32),
        compiler_params=pltpu.CompilerParams(
            fuse_transposed_lhs_in_matmul=True),
    )(table_t, eye)


def _gather_sc(table2, idx):
    N = idx.shape[0]
    b_per_w = N // NW
    n_chunks = b_per_w // CHUNK
    mesh = plsc.VectorSubcoreMesh(core_axis_name="c", subcore_axis_name="s")

    @functools.partial(
        pl.kernel,
        mesh=mesh,
        out_type=jax.ShapeDtypeStruct((N, 2 * E), jnp.float32),
        scratch_types=[
            pltpu.VMEM((b_per_w,), jnp.int32),
            pltpu.VMEM((CHUNK, 2 * E), jnp.float32),
            pltpu.SemaphoreType.DMA,
        ],
    )
    def k(table_hbm, idx_hbm, out_hbm, idx_v, rows_v, sem):
        c = lax.axis_index("c")
        s = lax.axis_index("s")
        wid = s * 2 + c
        base = wid * b_per_w
        pltpu.sync_copy(idx_hbm.at[pl.ds(base, b_per_w)], idx_v)

        def body(j, carry):
            off = j * CHUNK
            copies = []
            for t in range(K):
                copies.append(
                    pltpu.async_copy(
                        table_hbm.at[idx_v.at[pl.ds(off + t * CH, CH)]],
                        rows_v.at[pl.ds(t * CH, CH)],
                        sem,
                    )
                )
            for cp in copies:
                cp.wait()
            pltpu.sync_copy(rows_v, out_hbm.at[pl.ds(base + off, CHUNK)])
            return carry

        lax.fori_loop(0, n_chunks, body, 0)

    return k(table2, idx)


def _project_tc(x3, w2, par, S, B):
    def body(x_ref, w_ref, p_ref, o_ref):
        z = lax.dot_general(
            w_ref[...], x_ref[0], (((1,), (1,)), ((), ())),
            preferred_element_type=jnp.float32,
        )
        zlo = z[:E, :]
        zhi = z[E:, :]
        o_ref[0] = zlo + p_ref[0] * (zhi - zlo)

    return pl.pallas_call(
        body,
        grid=(S,),
        in_specs=[
            pl.BlockSpec((1, B, 2 * E), lambda s: (s, 0, 0)),
            pl.BlockSpec((2 * E, 2 * E), lambda s: (0, 0)),
            pl.BlockSpec((1, 1, B), lambda s: (s, 0, 0)),
        ],
        out_specs=pl.BlockSpec((1, E, B), lambda s: (s, 0, 0)),
        out_shape=jax.ShapeDtypeStruct((S, E, B), jnp.float32),
    )(x3, w2, par)


def kernel(sent1, sent2, embedding_table, W):
    B, S = sent1.shape
    H = VB // 2
    table2 = _pack_table_tc(embedding_table.T, jnp.eye(E, dtype=jnp.float32))
    i1 = sent1.T.astype(jnp.int32)
    i2 = sent2.T.astype(jnp.int32)
    sh = VB.bit_length() - 1

    def pack_idx(i):
        blk = i >> sh
        o = i & (VB - 1)
        return (blk << (sh - 1)) | (o & (H - 1)), (o >> (sh - 1)) & 1

    r1, h1 = pack_idx(i1)
    r2, h2 = pack_idx(i2)
    zero = jnp.zeros((E, E), jnp.float32)
    w2 = jnp.block([[W, zero], [zero, W]])
    p1 = h1.astype(jnp.float32).reshape(S, 1, B)
    p2 = h2.astype(jnp.float32).reshape(S, 1, B)
    g1 = _gather_sc(table2, r1.reshape(-1))
    g2 = _gather_sc(table2, r2.reshape(-1))
    y1 = _project_tc(g1.reshape(S, B, 2 * E), w2, p1, S, B)
    y2 = _project_tc(g2.reshape(S, B, 2 * E), w2, p2, S, B)
    s1 = y1.transpose(2, 0, 1)
    s2 = y2.transpose(2, 0, 1)
    return (s1, s2)
